# Initial kernel scaffold; baseline (speedup 1.0000x reference)
import jax
import jax.numpy as jnp
from jax.experimental import pallas as pl


def kernel(latent, weight):
    raise NotImplementedError("write your pallas kernel here")


# speedup vs baseline: 1.4665x; 1.4665x over previous
import functools

import jax
import jax.numpy as jnp
from jax import lax
from jax.experimental import pallas as pl
from jax.experimental.pallas import tpu as pltpu
from jax.experimental.pallas import tpu_sc as plsc

_NUM_CODES = 8192
_DIM = 256
_NUM_TOKENS = 8192
_M_BLK = 2048
_I32_MAX = jnp.iinfo(jnp.int32).max
_WINDOWS = ((0, 2048), (2048, 4096), (4096, 6144), (6144, 8192))


def _dist_argmin_body(zsq_ref, esq_ref, xb_ref, w_ref, idx_ref, minval_ref):
    acc = None
    aidx = None
    for (s, e) in _WINDOWS:
        dot2 = lax.dot_general(
            xb_ref[...].astype(jnp.float32), w_ref[s:e, :],
            dimension_numbers=(((1,), (1,)), ((), ())),
            preferred_element_type=jnp.float32,
        )
        dist = (zsq_ref[...] + esq_ref[0:1, s:e]) - dot2
        lmin = jnp.min(dist, axis=1, keepdims=True)
        cols = lax.broadcasted_iota(jnp.int32, dist.shape, 1).astype(
            jnp.float32)
        lidx = jnp.min(jnp.where(dist == lmin, cols, jnp.inf), axis=1,
                       keepdims=True) + float(s)
        if acc is None:
            acc = lmin.astype(jnp.bfloat16).astype(jnp.float32)
            aidx = lidx
        else:
            upd = lmin < acc
            aidx = jnp.where(upd, lidx, aidx)
            acc = jnp.where(
                upd, lmin.astype(jnp.bfloat16).astype(jnp.float32), acc)
    idx_ref[...] = aidx.astype(jnp.int32)
    minval_ref[...] = acc


_dist_argmin = pl.pallas_call(
    _dist_argmin_body,
    grid=(_NUM_TOKENS // _M_BLK,),
    in_specs=[
        pl.BlockSpec((_M_BLK, 1), lambda i: (i, 0)),
        pl.BlockSpec((1, _NUM_CODES), lambda i: (0, 0)),
        pl.BlockSpec((_M_BLK, _DIM), lambda i: (i, 0)),
        pl.BlockSpec((_NUM_CODES, _DIM), lambda i: (0, 0)),
    ],
    out_specs=[
        pl.BlockSpec((_M_BLK, 1), lambda i: (i, 0)),
        pl.BlockSpec((_M_BLK, 1), lambda i: (i, 0)),
    ],
    out_shape=[
        jax.ShapeDtypeStruct((_NUM_TOKENS, 1), jnp.int32),
        jax.ShapeDtypeStruct((_NUM_TOKENS, 1), jnp.float32),
    ],
)

_NC = 2
_NS = 16
_NW = _NC * _NS
_B_PER_W = _NUM_TOKENS // _NW


@functools.cache
def _make_gather_rows():
    @functools.partial(
        pl.kernel,
        mesh=plsc.VectorSubcoreMesh(core_axis_name="c", subcore_axis_name="s"),
        out_type=jax.ShapeDtypeStruct((_NUM_TOKENS, _DIM), jnp.float32),
        scratch_types=[
            pltpu.VMEM((_B_PER_W,), jnp.int32),
            pltpu.VMEM((_B_PER_W, _DIM), jnp.float32),
            pltpu.SemaphoreType.DMA,
        ],
    )
    def _gather_rows(idx_hbm, table_hbm, out_hbm, idx_v, rows_v, sem):
        wid = lax.axis_index("s") * _NC + lax.axis_index("c")
        base = wid * _B_PER_W
        pltpu.sync_copy(idx_hbm.at[pl.ds(base, _B_PER_W)], idx_v)
        pltpu.async_copy(table_hbm.at[idx_v], rows_v, sem).wait()
        pltpu.sync_copy(rows_v, out_hbm.at[pl.ds(base, _B_PER_W)])

    return _gather_rows


def kernel(latent, weight):
    x = jnp.transpose(latent, (0, 2, 3, 1))
    b, h, w, d = x.shape
    flat = x.reshape(-1, d)
    zsq = jnp.sum(x ** 2, axis=3).reshape(-1, 1)
    esq = jnp.sum(weight ** 2, axis=1)[None, :]
    xb = (2.0 * flat).astype(jnp.bfloat16)
    idx2, minval2 = _dist_argmin(zsq, esq, xb, weight)
    quantized_flat = _make_gather_rows()(idx2.reshape(-1), weight)
    loss = jnp.sum(minval2) / float(_NUM_TOKENS * _DIM)
    quantized = quantized_flat.reshape(b, h, w, d)
    out = jnp.transpose(quantized, (0, 3, 1, 2))
    return (out, loss)
